# baseline (speedup 1.0000x reference)
import functools

import jax
import jax.numpy as jnp
from jax import lax
from jax.experimental import pallas as pl
from jax.experimental.pallas import tpu as pltpu
from jax.experimental.pallas import tpu_sc as plsc

_K = 0


def kernel(u):
    B, S, D = u.shape
    NC, NS = 2, 16
    NW = NC * NS
    chunk = (B * D) // NW

    mesh = plsc.VectorSubcoreMesh(core_axis_name="c", subcore_axis_name="s")

    @functools.partial(
        pl.kernel,
        mesh=mesh,
        out_type=jax.ShapeDtypeStruct((B * D,), u.dtype),
        scratch_types=[
            pltpu.VMEM((chunk,), u.dtype),
            pltpu.SemaphoreType.DMA,
        ],
    )
    def sc_row_gather(u_hbm, o_hbm, buf, sem):
        wid = lax.axis_index("s") * NC + lax.axis_index("c")
        base = wid * chunk
        b = base // D
        off = base - b * D
        src = b * (S * D) + _K * D + off
        pltpu.async_copy(u_hbm.at[pl.ds(src, chunk)], buf, sem).wait()
        pltpu.async_copy(buf, o_hbm.at[pl.ds(base, chunk)], sem).wait()

    flat = sc_row_gather(u.reshape(B * S * D))
    return flat.reshape(B, 1, D)


# speedup vs baseline: 5.8305x; 5.8305x over previous
import functools

import jax
import jax.numpy as jnp
from jax import lax
from jax.experimental import pallas as pl
from jax.experimental.pallas import tpu as pltpu
from jax.experimental.pallas import tpu_sc as plsc

_K = 0


def kernel(u):
    B, S, D = u.shape
    NC, NS = 2, 16
    NW = NC * NS
    chunk = (B * D) // NW
    per_row = D // chunk

    mesh = plsc.VectorSubcoreMesh(core_axis_name="c", subcore_axis_name="s")

    @functools.partial(
        pl.kernel,
        mesh=mesh,
        out_type=jax.ShapeDtypeStruct((B, 1, D), u.dtype),
        scratch_types=[
            pltpu.VMEM((chunk,), u.dtype),
            pltpu.SemaphoreType.DMA,
        ],
    )
    def sc_row_gather(u_hbm, o_hbm, buf, sem):
        wid = lax.axis_index("s") * NC + lax.axis_index("c")
        b = wid // per_row
        off = (wid - b * per_row) * chunk
        pltpu.async_copy(u_hbm.at[b, _K, pl.ds(off, chunk)], buf, sem).wait()
        pltpu.async_copy(buf, o_hbm.at[b, 0, pl.ds(off, chunk)], sem).wait()

    return sc_row_gather(u)


# speedup vs baseline: 6.2125x; 1.0655x over previous
import functools

import jax
import jax.numpy as jnp
from jax import lax
from jax.experimental import pallas as pl
from jax.experimental.pallas import tpu as pltpu
from jax.experimental.pallas import tpu_sc as plsc

_K = 0


def kernel(u):
    B, S, D = u.shape

    mesh = plsc.ScalarSubcoreMesh(axis_name="c", num_cores=2)

    @functools.partial(
        pl.kernel,
        mesh=mesh,
        out_type=jax.ShapeDtypeStruct((B, 1, D), u.dtype),
        scratch_types=[pltpu.SemaphoreType.DMA],
    )
    def sc_row_gather(u_hbm, o_hbm, sem):
        cid = lax.axis_index("c")
        b0 = cid * (B // 2)
        c1 = pltpu.make_async_copy(u_hbm.at[b0, _K], o_hbm.at[b0, 0], sem)
        c2 = pltpu.make_async_copy(u_hbm.at[b0 + 1, _K], o_hbm.at[b0 + 1, 0], sem)
        c1.start()
        c2.start()
        c1.wait()
        c2.wait()

    return sc_row_gather(u)


# speedup vs baseline: 6.5311x; 1.0513x over previous
import functools

import jax
import jax.numpy as jnp
from jax import lax
from jax.experimental import pallas as pl
from jax.experimental.pallas import tpu as pltpu
from jax.experimental.pallas import tpu_sc as plsc

_K = 0


def kernel(u):
    B, S, D = u.shape

    mesh = plsc.ScalarSubcoreMesh(axis_name="c", num_cores=1)

    @functools.partial(
        pl.kernel,
        mesh=mesh,
        out_type=jax.ShapeDtypeStruct((B, 1, D), u.dtype),
        scratch_types=[pltpu.SemaphoreType.DMA],
    )
    def sc_row_gather(u_hbm, o_hbm, sem):
        copies = [
            pltpu.make_async_copy(u_hbm.at[b, _K], o_hbm.at[b, 0], sem)
            for b in range(B)
        ]
        for c in copies:
            c.start()
        for c in copies:
            c.wait()

    return sc_row_gather(u)


# speedup vs baseline: 6.5836x; 1.0080x over previous
import functools

import jax
import jax.numpy as jnp
from jax import lax
from jax.experimental import pallas as pl
from jax.experimental.pallas import tpu as pltpu
from jax.experimental.pallas import tpu_sc as plsc

_K = 0


def kernel(u):
    B, S, D = u.shape

    mesh = plsc.ScalarSubcoreMesh(axis_name="c", num_cores=1)

    @functools.partial(
        pl.kernel,
        mesh=mesh,
        out_type=jax.ShapeDtypeStruct((B, 1, D), u.dtype),
        scratch_types=[pltpu.SemaphoreType.DMA],
    )
    def sc_row_gather(u_hbm, o_hbm, sem):
        copy = pltpu.make_async_copy(u_hbm.at[:, _K], o_hbm.at[:, 0], sem)
        copy.start()
        copy.wait()

    return sc_row_gather(u)
